# baseline (speedup 1.0000x reference)
import functools

import jax
import jax.numpy as jnp
from jax import lax
from jax.experimental import pallas as pl
from jax.experimental.pallas import tpu as pltpu
from jax.experimental.pallas import tpu_sc as plsc
from jax._src.pallas import mpmd

MEM = 10000
D = 3072
B = 1024
R = 512
NC = 2
NS = 16
NW = NC * NS
L = 16

_MESH = plsc.VectorSubcoreMesh(core_axis_name="c", subcore_axis_name="s")
_SC_PARAMS = pltpu.CompilerParams(needs_layout_passes=False)


_N_COPY_DMA = 10
_COPY_ROWS = MEM // _N_COPY_DMA


def _copy_body(src_ref, dst_ref, sem):
    for k in range(_N_COPY_DMA):
        pltpu.make_async_copy(
            src_ref.at[pl.ds(k * _COPY_ROWS, _COPY_ROWS)],
            dst_ref.at[pl.ds(k * _COPY_ROWS, _COPY_ROWS)],
            sem,
        ).start()
    for k in range(_N_COPY_DMA):
        pltpu.make_async_copy(
            src_ref.at[pl.ds(k * _COPY_ROWS, _COPY_ROWS)],
            dst_ref.at[pl.ds(k * _COPY_ROWS, _COPY_ROWS)],
            sem,
        ).wait()


@jax.jit
def _tc_copy(buf):
    return pl.pallas_call(
        _copy_body,
        out_shape=jax.ShapeDtypeStruct((MEM, D), jnp.float32),
        in_specs=[pl.BlockSpec(memory_space=pl.ANY)],
        out_specs=pl.BlockSpec(memory_space=pl.ANY),
        scratch_shapes=[pltpu.SemaphoreType.DMA],
    )(buf)


def _update_body(img_in, x, y, idx, blab,
                 img_out, nlab,
                 idx_v, y_v, w_v, lab_v, stage,
                 w_sh,
                 sem):
    c = lax.axis_index("c")
    s = lax.axis_index("s")
    wid = s * NC + c

    pltpu.sync_copy(idx, idx_v)
    lanes = lax.iota(jnp.int32, L)

    @pl.when(s == 0)
    def _build_map():
        def setw(ci, carry):
            iv = idx_v[pl.ds(ci * L, L)]
            bids = ci * L + lanes
            for l in range(L):
                plsc.store_scatter(w_v, [iv], bids, mask=lanes == l)
            return carry

        lax.fori_loop(0, B // L, setw, 0)
        pltpu.sync_copy(w_v, w_sh)

    plsc.subcore_barrier()
    pltpu.sync_copy(w_sh, w_v)

    per = B // NW
    base = wid * per
    for h in range(per // L):
        dv = idx_v[pl.ds(base + h * L, L)]
        srcs = plsc.load_gather(w_v, [dv])
        pltpu.async_copy(x.at[srcs], stage, sem).wait()
        pltpu.async_copy(stage, img_out.at[dv], sem).wait()

    @pl.when(jnp.logical_and(s == 0, c == 0))
    def _labels():
        pltpu.sync_copy(blab, lab_v)
        pltpu.sync_copy(y, y_v)

        def setl(ci, carry):
            iv = idx_v[pl.ds(ci * L, L)]
            yv = y_v[pl.ds(ci * L, L)]
            for l in range(L):
                plsc.store_scatter(lab_v, [iv], yv, mask=lanes == l)
            return carry

        lax.fori_loop(0, B // L, setl, 0)
        pltpu.sync_copy(lab_v, nlab)


_sc_update = mpmd._mpmd_map(
    [(_MESH, _update_body)],
    out_types=[
        jax.ShapeDtypeStruct((MEM, D), jnp.float32),
        jax.ShapeDtypeStruct((MEM,), jnp.int32),
    ],
    input_output_aliases={0: 0},
    compiler_params=_SC_PARAMS,
    scratch_types=[
        pltpu.VMEM((B,), jnp.int32),
        pltpu.VMEM((B,), jnp.int32),
        pltpu.VMEM((MEM,), jnp.int32),
        pltpu.VMEM((MEM,), jnp.int32),
        pltpu.VMEM((L, D), jnp.float32),
        pltpu.VMEM_SHARED((MEM,), jnp.int32),
        pltpu.SemaphoreType.DMA,
    ],
)


def _retrieve_body(img, nlab, ridx,
                   rimg, rlab,
                   ridx_v, rlab_v, lab_v, stage,
                   sem):
    c = lax.axis_index("c")
    s = lax.axis_index("s")
    wid = s * NC + c
    per = R // NW

    pltpu.sync_copy(ridx.at[pl.ds(wid * per, per)], ridx_v)
    rv = ridx_v[...]
    pltpu.async_copy(img.at[rv], stage, sem).wait()
    pltpu.sync_copy(stage, rimg.at[pl.ds(wid * per, per)])

    @pl.when(jnp.logical_and(s == 0, c == 0))
    def _labels():
        pltpu.sync_copy(nlab, lab_v)
        pltpu.sync_copy(ridx, rlab_v)

        def lchunk(ci, carry):
            rr = rlab_v[pl.ds(ci * L, L)]
            lv = plsc.load_gather(lab_v, [rr])
            rlab_v[pl.ds(ci * L, L)] = lv
            return carry

        lax.fori_loop(0, R // L, lchunk, 0)
        pltpu.sync_copy(rlab_v, rlab)


_sc_retrieve = mpmd._mpmd_map(
    [(_MESH, _retrieve_body)],
    out_types=[
        jax.ShapeDtypeStruct((R, D), jnp.float32),
        jax.ShapeDtypeStruct((R,), jnp.int32),
    ],
    compiler_params=_SC_PARAMS,
    scratch_types=[
        pltpu.VMEM((R // NW,), jnp.int32),
        pltpu.VMEM((R,), jnp.int32),
        pltpu.VMEM((MEM,), jnp.int32),
        pltpu.VMEM((R // NW, D), jnp.float32),
        pltpu.SemaphoreType.DMA,
    ],
)


def kernel(buffer_img, buffer_label, x, y, idx, retrieve_idx):
    img2 = buffer_img.reshape(MEM, D)
    x2 = x.reshape(B, D)
    y32 = y.astype(jnp.int32)
    idx32 = idx.astype(jnp.int32)
    ridx32 = retrieve_idx.astype(jnp.int32)
    blab32 = buffer_label.astype(jnp.int32)

    img0 = _tc_copy(img2)
    new_img2, new_label = _sc_update(img0, x2, y32, idx32, blab32)
    ret_img2, ret_label = _sc_retrieve(new_img2, new_label, ridx32)

    new_img = new_img2.reshape(MEM, 3, 32, 32)
    ret_img = ret_img2.reshape(R, 3, 32, 32)
    return (new_img,
            new_label.astype(buffer_label.dtype),
            ret_img,
            ret_label.astype(buffer_label.dtype))


# speedup vs baseline: 7.7117x; 7.7117x over previous
import jax
import jax.numpy as jnp
from jax import lax
from jax.experimental import pallas as pl
from jax.experimental.pallas import tpu as pltpu
from jax.experimental.pallas import tpu_sc as plsc
from jax._src.pallas import mpmd

MEM = 10000
D = 3072
B = 1024
R = 512
NC = 2
NS = 16
NW = NC * NS
L = 16

NCHUNK = MEM // L
CH_PER = -(-NCHUNK // NW)

_MESH = plsc.VectorSubcoreMesh(core_axis_name="c", subcore_axis_name="s")
_SC_PARAMS = pltpu.CompilerParams(needs_layout_passes=False)


def _body(img, x, y, idx, blab, ridx,
          nimg, nlab, rimg, rlab,
          idx_v, w_v, stage, ridx_v, lab_v, y_v, ridx_all, rlab_v,
          w_sh,
          rsem, wsem, ssem):
    c = lax.axis_index("c")
    s = lax.axis_index("s")
    wid = s * NC + c
    lanes = lax.iota(jnp.int32, L)

    pltpu.sync_copy(idx, idx_v)

    @pl.when(s == 0)
    def _build_map():
        def mset(i, carry):
            w_v[pl.ds(i * L, L)] = jnp.full((L,), -1, jnp.int32)
            return carry

        lax.fori_loop(0, MEM // L, mset, 0)

        def setw(ci, carry):
            iv = idx_v[pl.ds(ci * L, L)]
            bids = ci * L + lanes
            for l in range(L):
                plsc.store_scatter(w_v, [iv], bids, mask=lanes == l)
            return carry

        lax.fori_loop(0, B // L, setw, 0)
        pltpu.sync_copy(w_v, w_sh)

    start = wid * CH_PER

    def _rd(k, p):
        return pltpu.make_async_copy(
            img.at[pl.ds((start + k) * L, L)], stage.at[p], rsem)

    def _wr(k, p):
        return pltpu.make_async_copy(
            stage.at[p], nimg.at[pl.ds((start + k) * L, L)], wsem)

    def _valid(k):
        return start + k < NCHUNK

    _rd(0, 0).start()
    for k in range(CH_PER):
        p = k % 2
        pl.when(_valid(k))((lambda k=k, p=p: _rd(k, p).wait()))
        if k >= 1:
            pl.when(_valid(k - 1))(
                (lambda k=k, p=p: _wr(k - 1, 1 - p).wait()))
        if k + 1 < CH_PER:
            pl.when(_valid(k + 1))(
                (lambda k=k, p=p: _rd(k + 1, 1 - p).start()))
        pl.when(_valid(k))((lambda k=k, p=p: _wr(k, p).start()))
    pl.when(_valid(CH_PER - 1))(
        (lambda: _wr(CH_PER - 1, (CH_PER - 1) % 2).wait()))

    plsc.subcore_barrier()
    pltpu.sync_copy(w_sh, w_v)

    for k in range(CH_PER):
        @pl.when(_valid(k))
        def _(k=k):
            rows = (start + k) * L + lanes
            wv = plsc.load_gather(w_v, [rows])
            upd = wv >= 0

            @pl.when(jnp.any(upd))
            def _():
                f = plsc.all_reduce_ffs(upd)
                row_f = (start + k) * L + f
                wv_f = plsc.load_gather(w_v, [row_f])
                src = jnp.where(upd, wv, wv_f)
                dst = jnp.where(upd, rows, row_f)
                pltpu.async_copy(x.at[src], stage.at[0], ssem).wait()
                pltpu.async_copy(stage.at[0], nimg.at[dst], ssem).wait()

    rbase = wid * (R // NW)
    pltpu.sync_copy(ridx.at[pl.ds(rbase, R // NW)], ridx_v)
    rv = ridx_v[...]
    pltpu.async_copy(img.at[rv], stage.at[0], ssem).wait()
    pltpu.async_copy(stage.at[0], rimg.at[pl.ds(rbase, R // NW)], ssem).wait()
    wr = plsc.load_gather(w_v, [rv])
    updr = wr >= 0

    @pl.when(jnp.any(updr))
    def _ret_overwrite():
        f = plsc.all_reduce_ffs(updr)
        rv_f = plsc.load_gather(ridx_v, [f])
        wr_f = plsc.load_gather(w_v, [rv_f])
        src = jnp.where(updr, wr, wr_f)
        dst = jnp.where(updr, rbase + lanes, rbase + f)
        pltpu.async_copy(x.at[src], stage.at[1], ssem).wait()
        pltpu.async_copy(stage.at[1], rimg.at[dst], ssem).wait()

    @pl.when(jnp.logical_and(s == 0, c == 0))
    def _labels():
        pltpu.sync_copy(blab, lab_v)
        pltpu.sync_copy(y, y_v)

        def setl(ci, carry):
            iv = idx_v[pl.ds(ci * L, L)]
            yv = y_v[pl.ds(ci * L, L)]
            for l in range(L):
                plsc.store_scatter(lab_v, [iv], yv, mask=lanes == l)
            return carry

        lax.fori_loop(0, B // L, setl, 0)
        pltpu.sync_copy(lab_v, nlab)

        pltpu.sync_copy(ridx, ridx_all)

        def rl(ci, carry):
            rr = ridx_all[pl.ds(ci * L, L)]
            rlab_v[pl.ds(ci * L, L)] = plsc.load_gather(lab_v, [rr])
            return carry

        lax.fori_loop(0, R // L, rl, 0)
        pltpu.sync_copy(rlab_v, rlab)


_sc_all = mpmd._mpmd_map(
    [(_MESH, _body)],
    out_types=[
        jax.ShapeDtypeStruct((MEM, D), jnp.float32),
        jax.ShapeDtypeStruct((MEM,), jnp.int32),
        jax.ShapeDtypeStruct((R, D), jnp.float32),
        jax.ShapeDtypeStruct((R,), jnp.int32),
    ],
    compiler_params=_SC_PARAMS,
    scratch_types=[
        pltpu.VMEM((B,), jnp.int32),
        pltpu.VMEM((MEM,), jnp.int32),
        pltpu.VMEM((2, L, D), jnp.float32),
        pltpu.VMEM((R // NW,), jnp.int32),
        pltpu.VMEM((MEM,), jnp.int32),
        pltpu.VMEM((B,), jnp.int32),
        pltpu.VMEM((R,), jnp.int32),
        pltpu.VMEM((R,), jnp.int32),
        pltpu.VMEM_SHARED((MEM,), jnp.int32),
        pltpu.SemaphoreType.DMA,
        pltpu.SemaphoreType.DMA,
        pltpu.SemaphoreType.DMA,
    ],
)


def kernel(buffer_img, buffer_label, x, y, idx, retrieve_idx):
    img2 = buffer_img.reshape(MEM, D)
    x2 = x.reshape(B, D)
    y32 = y.astype(jnp.int32)
    idx32 = idx.astype(jnp.int32)
    ridx32 = retrieve_idx.astype(jnp.int32)
    blab32 = buffer_label.astype(jnp.int32)

    new_img2, new_label, ret_img2, ret_label = _sc_all(
        img2, x2, y32, idx32, blab32, ridx32)

    new_img = new_img2.reshape(MEM, 3, 32, 32)
    ret_img = ret_img2.reshape(R, 3, 32, 32)
    return (new_img,
            new_label.astype(buffer_label.dtype),
            ret_img,
            ret_label.astype(buffer_label.dtype))


# speedup vs baseline: 11.1646x; 1.4477x over previous
import jax
import jax.numpy as jnp
from jax import lax
from jax.experimental import pallas as pl
from jax.experimental.pallas import tpu as pltpu
from jax.experimental.pallas import tpu_sc as plsc
from jax._src.pallas import mpmd

MEM = 10000
D = 3072
B = 1024
R = 512
NC = 2
NS = 16
NW = NC * NS
L = 16

_MESH = plsc.VectorSubcoreMesh(core_axis_name="c", subcore_axis_name="s")
_SC_PARAMS = pltpu.CompilerParams(needs_layout_passes=False)


def _copy_body(src_ref, dst_ref):
    dst_ref[...] = src_ref[...]


_COPY_BLK = 400


@jax.jit
def _tc_copy(buf):
    return pl.pallas_call(
        _copy_body,
        out_shape=jax.ShapeDtypeStruct((MEM, D), jnp.float32),
        grid=(MEM // _COPY_BLK,),
        in_specs=[pl.BlockSpec((_COPY_BLK, D), lambda i: (i, 0))],
        out_specs=pl.BlockSpec((_COPY_BLK, D), lambda i: (i, 0)),
    )(buf)


def _update_body(img_in, x, y, idx, blab, ridx,
                 img_out, nlab, rimg, rlab,
                 idx_v, w_v, stage, ridx_v, lab_v, y_v, ridx_all, rlab_v,
                 w_sh,
                 sem, ssem):
    c = lax.axis_index("c")
    s = lax.axis_index("s")
    wid = s * NC + c
    lanes = lax.iota(jnp.int32, L)

    pltpu.sync_copy(idx, idx_v)

    @pl.when(s == 0)
    def _build_map():
        def mset(i, carry):
            w_v[pl.ds(i * L, L)] = jnp.full((L,), -1, jnp.int32)
            return carry

        lax.fori_loop(0, MEM // L, mset, 0)

        def setw(ci, carry):
            iv = idx_v[pl.ds(ci * L, L)]
            bids = ci * L + lanes
            for l in range(L):
                plsc.store_scatter(w_v, [iv], bids, mask=lanes == l)
            return carry

        lax.fori_loop(0, B // L, setw, 0)
        pltpu.sync_copy(w_v, w_sh)

    plsc.subcore_barrier()
    pltpu.sync_copy(w_sh, w_v)

    per = B // NW
    base = wid * per
    for h in range(per // L):
        dv = idx_v[pl.ds(base + h * L, L)]
        srcs = plsc.load_gather(w_v, [dv])
        pltpu.async_copy(x.at[srcs], stage.at[h], sem).wait()
        pltpu.async_copy(stage.at[h], img_out.at[dv], sem).wait()

    rbase = wid * (R // NW)
    pltpu.sync_copy(ridx.at[pl.ds(rbase, R // NW)], ridx_v)
    rv = ridx_v[...]
    pltpu.async_copy(img_in.at[rv], stage.at[0], ssem).wait()
    pltpu.async_copy(stage.at[0], rimg.at[pl.ds(rbase, R // NW)], ssem).wait()
    wr = plsc.load_gather(w_v, [rv])
    updr = wr >= 0

    @pl.when(jnp.any(updr))
    def _ret_patch():
        f = plsc.all_reduce_ffs(updr)
        rv_f = plsc.load_gather(ridx_v, [f])
        wr_f = plsc.load_gather(w_v, [rv_f])
        srcs = jnp.where(updr, wr, wr_f)
        dst = jnp.where(updr, rbase + lanes, rbase + f)
        pltpu.async_copy(x.at[srcs], stage.at[1], ssem).wait()
        pltpu.async_copy(stage.at[1], rimg.at[dst], ssem).wait()

    @pl.when(jnp.logical_and(s == 0, c == 0))
    def _labels():
        pltpu.sync_copy(blab, lab_v)
        pltpu.sync_copy(y, y_v)

        def setl(ci, carry):
            iv = idx_v[pl.ds(ci * L, L)]
            yv = y_v[pl.ds(ci * L, L)]
            for l in range(L):
                plsc.store_scatter(lab_v, [iv], yv, mask=lanes == l)
            return carry

        lax.fori_loop(0, B // L, setl, 0)
        pltpu.sync_copy(lab_v, nlab)

        pltpu.sync_copy(ridx, ridx_all)

        def rl(ci, carry):
            rr = ridx_all[pl.ds(ci * L, L)]
            rlab_v[pl.ds(ci * L, L)] = plsc.load_gather(lab_v, [rr])
            return carry

        lax.fori_loop(0, R // L, rl, 0)
        pltpu.sync_copy(rlab_v, rlab)


_sc_update = mpmd._mpmd_map(
    [(_MESH, _update_body)],
    out_types=[
        jax.ShapeDtypeStruct((MEM, D), jnp.float32),
        jax.ShapeDtypeStruct((MEM,), jnp.int32),
        jax.ShapeDtypeStruct((R, D), jnp.float32),
        jax.ShapeDtypeStruct((R,), jnp.int32),
    ],
    input_output_aliases={0: 0},
    compiler_params=_SC_PARAMS,
    scratch_types=[
        pltpu.VMEM((B,), jnp.int32),
        pltpu.VMEM((MEM,), jnp.int32),
        pltpu.VMEM((2, L, D), jnp.float32),
        pltpu.VMEM((R // NW,), jnp.int32),
        pltpu.VMEM((MEM,), jnp.int32),
        pltpu.VMEM((B,), jnp.int32),
        pltpu.VMEM((R,), jnp.int32),
        pltpu.VMEM((R,), jnp.int32),
        pltpu.VMEM_SHARED((MEM,), jnp.int32),
        pltpu.SemaphoreType.DMA,
        pltpu.SemaphoreType.DMA,
    ],
)


def kernel(buffer_img, buffer_label, x, y, idx, retrieve_idx):
    img2 = buffer_img.reshape(MEM, D)
    x2 = x.reshape(B, D)
    y32 = y.astype(jnp.int32)
    idx32 = idx.astype(jnp.int32)
    ridx32 = retrieve_idx.astype(jnp.int32)
    blab32 = buffer_label.astype(jnp.int32)

    img0 = _tc_copy(img2)
    new_img2, new_label, ret_img2, ret_label = _sc_update(
        img0, x2, y32, idx32, blab32, ridx32)

    new_img = new_img2.reshape(MEM, 3, 32, 32)
    ret_img = ret_img2.reshape(R, 3, 32, 32)
    return (new_img,
            new_label.astype(buffer_label.dtype),
            ret_img,
            ret_label.astype(buffer_label.dtype))
